# baseline (speedup 1.0000x reference)
import functools

import jax
import jax.numpy as jnp
from jax import lax
from jax.experimental import pallas as pl
from jax.experimental.pallas import tpu as pltpu
from jax.experimental.pallas import tpu_sc as plsc

B = 16384
D = 128
NC = 2
NS = 16
NW = NC * NS
BPW = B // NW
CHUNK = 64
NCHUNK = BPW // CHUNK

_mesh = plsc.VectorSubcoreMesh(core_axis_name="c", subcore_axis_name="s")


@functools.partial(
    pl.kernel,
    mesh=_mesh,
    out_type=jax.ShapeDtypeStruct((B, D), jnp.float32),
    scratch_types=[
        pltpu.VMEM((BPW,), jnp.int32),
        pltpu.VMEM((NCHUNK, CHUNK, D), jnp.float32),
    ]
    + [pltpu.SemaphoreType.DMA] * (NCHUNK + 1),
)
def _emb_lookup(t_hbm, table_hbm, out_hbm, idx_v, rows_v, *sems):
    gsems, ssem = sems[:NCHUNK], sems[NCHUNK]
    wid = lax.axis_index("s") * NC + lax.axis_index("c")
    base = wid * BPW
    pltpu.sync_copy(t_hbm.at[pl.ds(base, BPW)], idx_v)
    gcps = [
        pltpu.async_copy(
            table_hbm.at[idx_v.at[pl.ds(j * CHUNK, CHUNK)]], rows_v.at[j], gsems[j]
        )
        for j in range(NCHUNK)
    ]
    scps = []
    for j in range(NCHUNK):
        gcps[j].wait()
        scps.append(
            pltpu.async_copy(
                rows_v.at[j], out_hbm.at[pl.ds(base + j * CHUNK, CHUNK)], ssem
            )
        )
    for cp in scps:
        cp.wait()


V = 1000
VP = 1024
NSLAB = VP // D
BT = 512
GT = B // BT


def _tc_body(t_ref, table_ref, out_ref):
    tb = t_ref[0, 0, :]
    a = tb >> 7
    c = tb & 127
    lanes = lax.broadcasted_iota(jnp.int32, (BT, D), 1)
    onehot_c = jnp.where(c[:, None] == lanes, 1.0, 0.0).astype(jnp.bfloat16)
    chunks = []
    for s in range(NSLAB):
        m = jnp.where(a[:, None] == s, 1.0, 0.0).astype(jnp.bfloat16)
        chunks.append(onehot_c * m)
    onehot = jnp.concatenate(chunks, axis=1)
    out_ref[...] = jnp.dot(
        onehot, table_ref[...], preferred_element_type=jnp.float32
    )


def _tc_lookup(t, table):
    table_p = jnp.pad(table, ((0, VP - V), (0, 0))).astype(jnp.bfloat16)
    return pl.pallas_call(
        _tc_body,
        grid=(GT,),
        in_specs=[
            pl.BlockSpec((1, 1, BT), lambda i: (i, 0, 0)),
            pl.BlockSpec((VP, D), lambda i: (0, 0)),
        ],
        out_specs=pl.BlockSpec((BT, D), lambda i: (i, 0)),
        out_shape=jax.ShapeDtypeStruct((B, D), jnp.float32),
    )(t.reshape(GT, 1, BT), table_p)


def kernel(t, table):
    return _tc_lookup(t, table)


# speedup vs baseline: 1.5740x; 1.5740x over previous
import functools

import jax
import jax.numpy as jnp
from jax import lax
from jax.experimental import pallas as pl
from jax.experimental.pallas import tpu as pltpu
from jax.experimental.pallas import tpu_sc as plsc

B = 16384
D = 128
NC = 2
NS = 16
NW = NC * NS
BPW = B // NW
CHUNK = 64
NCHUNK = BPW // CHUNK

_mesh = plsc.VectorSubcoreMesh(core_axis_name="c", subcore_axis_name="s")


@functools.partial(
    pl.kernel,
    mesh=_mesh,
    out_type=jax.ShapeDtypeStruct((B, D), jnp.float32),
    scratch_types=[
        pltpu.VMEM((BPW,), jnp.int32),
        pltpu.VMEM((NCHUNK, CHUNK, D), jnp.float32),
    ]
    + [pltpu.SemaphoreType.DMA] * (NCHUNK + 1),
)
def _emb_lookup(t_hbm, table_hbm, out_hbm, idx_v, rows_v, *sems):
    gsems, ssem = sems[:NCHUNK], sems[NCHUNK]
    wid = lax.axis_index("s") * NC + lax.axis_index("c")
    base = wid * BPW
    pltpu.sync_copy(t_hbm.at[pl.ds(base, BPW)], idx_v)
    gcps = [
        pltpu.async_copy(
            table_hbm.at[idx_v.at[pl.ds(j * CHUNK, CHUNK)]], rows_v.at[j], gsems[j]
        )
        for j in range(NCHUNK)
    ]
    scps = []
    for j in range(NCHUNK):
        gcps[j].wait()
        scps.append(
            pltpu.async_copy(
                rows_v.at[j], out_hbm.at[pl.ds(base + j * CHUNK, CHUNK)], ssem
            )
        )
    for cp in scps:
        cp.wait()


V = 1000
VP = 1024
NSLAB = VP // D
BT = 2048
GT = B // BT


def _tc_body(t_ref, table_ref, out_ref):
    tb = t_ref[0, 0, :]
    a = tb >> 7
    c = tb & 127
    lanes = lax.broadcasted_iota(jnp.int32, (BT, D), 1)
    onehot_c = jnp.where(c[:, None] == lanes, 1.0, 0.0).astype(jnp.bfloat16)
    chunks = []
    for s in range(NSLAB):
        m = jnp.where(a[:, None] == s, 1.0, 0.0).astype(jnp.bfloat16)
        chunks.append(onehot_c * m)
    onehot = jnp.concatenate(chunks, axis=1)
    out_ref[...] = jnp.dot(
        onehot, table_ref[...], preferred_element_type=jnp.float32
    )


def _tc_lookup(t, table):
    table_p = jnp.pad(table, ((0, VP - V), (0, 0))).astype(jnp.bfloat16)
    return pl.pallas_call(
        _tc_body,
        grid=(GT,),
        in_specs=[
            pl.BlockSpec((1, 1, BT), lambda i: (i, 0, 0)),
            pl.BlockSpec((VP, D), lambda i: (0, 0)),
        ],
        out_specs=pl.BlockSpec((BT, D), lambda i: (i, 0)),
        out_shape=jax.ShapeDtypeStruct((B, D), jnp.float32),
    )(t.reshape(GT, 1, BT), table_p)


def kernel(t, table):
    return _tc_lookup(t, table)


# speedup vs baseline: 1.6296x; 1.0354x over previous
import functools

import jax
import jax.numpy as jnp
from jax import lax
from jax.experimental import pallas as pl
from jax.experimental.pallas import tpu as pltpu
from jax.experimental.pallas import tpu_sc as plsc

B = 16384
D = 128
NC = 2
NS = 16
NW = NC * NS
BPW = B // NW
CHUNK = 64
NCHUNK = BPW // CHUNK

_mesh = plsc.VectorSubcoreMesh(core_axis_name="c", subcore_axis_name="s")


@functools.partial(
    pl.kernel,
    mesh=_mesh,
    out_type=jax.ShapeDtypeStruct((B, D), jnp.float32),
    scratch_types=[
        pltpu.VMEM((BPW,), jnp.int32),
        pltpu.VMEM((NCHUNK, CHUNK, D), jnp.float32),
    ]
    + [pltpu.SemaphoreType.DMA] * (NCHUNK + 1),
)
def _emb_lookup(t_hbm, table_hbm, out_hbm, idx_v, rows_v, *sems):
    gsems, ssem = sems[:NCHUNK], sems[NCHUNK]
    wid = lax.axis_index("s") * NC + lax.axis_index("c")
    base = wid * BPW
    pltpu.sync_copy(t_hbm.at[pl.ds(base, BPW)], idx_v)
    gcps = [
        pltpu.async_copy(
            table_hbm.at[idx_v.at[pl.ds(j * CHUNK, CHUNK)]], rows_v.at[j], gsems[j]
        )
        for j in range(NCHUNK)
    ]
    scps = []
    for j in range(NCHUNK):
        gcps[j].wait()
        scps.append(
            pltpu.async_copy(
                rows_v.at[j], out_hbm.at[pl.ds(base + j * CHUNK, CHUNK)], ssem
            )
        )
    for cp in scps:
        cp.wait()


V = 1000
VP = 1024
NSLAB = VP // D
BT = 4096
GT = B // BT


def _tc_body(t_ref, table_ref, out_ref):
    tb = t_ref[0, 0, :]
    a = tb >> 7
    c = tb & 127
    lanes = lax.broadcasted_iota(jnp.int32, (BT, D), 1)
    onehot_c = jnp.where(c[:, None] == lanes, 1.0, 0.0).astype(jnp.bfloat16)
    chunks = []
    for s in range(NSLAB):
        m = jnp.where(a[:, None] == s, 1.0, 0.0).astype(jnp.bfloat16)
        chunks.append(onehot_c * m)
    onehot = jnp.concatenate(chunks, axis=1)
    out_ref[...] = jnp.dot(
        onehot, table_ref[...], preferred_element_type=jnp.float32
    )


def _tc_lookup(t, table):
    table_p = jnp.pad(table, ((0, VP - V), (0, 0))).astype(jnp.bfloat16)
    return pl.pallas_call(
        _tc_body,
        grid=(GT,),
        in_specs=[
            pl.BlockSpec((1, 1, BT), lambda i: (i, 0, 0)),
            pl.BlockSpec((VP, D), lambda i: (0, 0)),
        ],
        out_specs=pl.BlockSpec((BT, D), lambda i: (i, 0)),
        out_shape=jax.ShapeDtypeStruct((B, D), jnp.float32),
    )(t.reshape(GT, 1, BT), table_p)


def kernel(t, table):
    return _tc_lookup(t, table)
